# Initial kernel scaffold; baseline (speedup 1.0000x reference)
import functools

import jax
import jax.numpy as jnp
from jax import lax
from jax.experimental import pallas as pl
from jax.experimental.pallas import tpu as pltpu
from jax.experimental.pallas import tpu_sc as plsc

V = 1000
N = 51200
NC, NS, L = 2, 16, 16
NW = NC * NS
RPW = N // NW
CH = 32
NCHUNK = RPW // CH


def _lse_body(tab_ref, out_ref):
    x = tab_ref[...]
    m = jnp.max(x, axis=1, keepdims=True)
    s = jnp.sum(jnp.exp(x - m), axis=1, keepdims=True)
    out_ref[...] = jnp.log(s) + m


def _row_lse(table):
    return pl.pallas_call(
        _lse_body,
        out_shape=jax.ShapeDtypeStruct((V, 1), jnp.float32),
    )(table)


def _fin_body(p_ref, out_ref):
    out_ref[...] = jnp.full((1, 1), jnp.sum(p_ref[...]) * (1.0 / N),
                            dtype=jnp.float32)


def _finalize(partials):
    return pl.pallas_call(
        _fin_body,
        out_shape=jax.ShapeDtypeStruct((1, 1), jnp.float32),
    )(partials)


def _sc_body(table_hbm, in_hbm, tg_hbm, lse_hbm, out_hbm, part_hbm,
             idx_v, tgt_v, lse_v, rows_v, acc_v, gsem):
    wid = lax.axis_index("s") * NC + lax.axis_index("c")
    base = wid * RPW

    pltpu.sync_copy(in_hbm.at[pl.ds(base, RPW)], idx_v)
    pltpu.sync_copy(tg_hbm.at[pl.ds(base, RPW)], tgt_v)
    pltpu.sync_copy(lse_hbm, lse_v)
    acc_v[...] = jnp.zeros((L,), jnp.float32)

    def chunk(c, carry):
        co = c * CH
        pltpu.async_copy(
            table_hbm.at[idx_v.at[pl.ds(co, CH)]], rows_v, gsem
        ).wait()
        for g in range(CH // L):
            rid = lax.iota(jnp.int32, L) + (g * L)
            ids = idx_v[pl.ds(co + g * L, L)]
            cid = tgt_v[pl.ds(co + g * L, L)]
            t_el = plsc.load_gather(rows_v, [rid, cid])
            lse_g = plsc.load_gather(lse_v, [ids])
            acc_v[...] = acc_v[...] + (lse_g - t_el)
        pltpu.sync_copy(rows_v, out_hbm.at[pl.ds(base + co, CH)])
        return carry

    lax.fori_loop(0, NCHUNK, chunk, 0)
    pltpu.sync_copy(acc_v, part_hbm.at[wid])


@functools.partial(
    pl.kernel,
    out_type=(
        jax.ShapeDtypeStruct((N, V), jnp.float32),
        jax.ShapeDtypeStruct((NW, L), jnp.float32),
    ),
    mesh=plsc.VectorSubcoreMesh(core_axis_name="c", subcore_axis_name="s"),
    scratch_types=[
        pltpu.VMEM((RPW,), jnp.int32),
        pltpu.VMEM((RPW,), jnp.int32),
        pltpu.VMEM((V,), jnp.float32),
        pltpu.VMEM((CH, V), jnp.float32),
        pltpu.VMEM((L,), jnp.float32),
        pltpu.SemaphoreType.DMA,
    ],
)
def _sc_gather(table_hbm, in_hbm, tg_hbm, lse_hbm, out_hbm, part_hbm,
               idx_v, tgt_v, lse_v, rows_v, acc_v, gsem):
    _sc_body(table_hbm, in_hbm, tg_hbm, lse_hbm, out_hbm, part_hbm,
             idx_v, tgt_v, lse_v, rows_v, acc_v, gsem)


def kernel(input_sequence, target_sequence, table):
    flat_in = input_sequence.reshape(-1)
    flat_tg = target_sequence.reshape(-1)
    lse = _row_lse(table)
    preds, partials = _sc_gather(table, flat_in, flat_tg,
                                 lse.reshape(-1))
    loss = _finalize(partials)[0, 0]
    return preds, loss


# speedup vs baseline: 1.6089x; 1.6089x over previous
import functools

import jax
import jax.numpy as jnp
from jax import lax
from jax.experimental import pallas as pl
from jax.experimental.pallas import tpu as pltpu
from jax.experimental.pallas import tpu_sc as plsc

V = 1000
N = 51200
NC, NS, L = 2, 16, 16
NW = NC * NS
RPW = N // NW
CH = 32
NCHUNK = RPW // CH


def _lse_body(tab_ref, out_ref):
    x = tab_ref[...]
    m = jnp.max(x, axis=1, keepdims=True)
    s = jnp.sum(jnp.exp(x - m), axis=1, keepdims=True)
    out_ref[...] = jnp.log(s) + m


def _row_lse(table):
    return pl.pallas_call(
        _lse_body,
        out_shape=jax.ShapeDtypeStruct((V, 1), jnp.float32),
    )(table)


def _fin_body(p_ref, out_ref):
    out_ref[...] = jnp.full((1, 1), jnp.sum(p_ref[...]) * (1.0 / N),
                            dtype=jnp.float32)


def _finalize(partials):
    return pl.pallas_call(
        _fin_body,
        out_shape=jax.ShapeDtypeStruct((1, 1), jnp.float32),
    )(partials)


def _sc_body(table_hbm, in_hbm, tg_hbm, lse_hbm,
             out_hbm, part_hbm,
             idx_v, tgt_v, lse_v, rows_v, acc_v, gsem):
    wid = lax.axis_index("s") * NC + lax.axis_index("c")
    base = wid * RPW

    pltpu.sync_copy(in_hbm.at[pl.ds(base, RPW)], idx_v)
    pltpu.sync_copy(tg_hbm.at[pl.ds(base, RPW)], tgt_v)
    pltpu.sync_copy(lse_hbm, lse_v)
    acc_v[...] = jnp.zeros((L,), jnp.float32)

    def chunk(c, carry):
        co = c * CH
        pltpu.async_copy(
            table_hbm.at[idx_v.at[pl.ds(co, CH)]], rows_v, gsem
        ).wait()
        for g in range(CH // L):
            rid = lax.iota(jnp.int32, L) + (g * L)
            ids = idx_v[pl.ds(co + g * L, L)]
            cid = tgt_v[pl.ds(co + g * L, L)]
            t_el = plsc.load_gather(rows_v, [rid, cid])
            lse_g = plsc.load_gather(lse_v, [ids])
            acc_v[...] = acc_v[...] + (lse_g - t_el)
        pltpu.sync_copy(rows_v, out_hbm.at[pl.ds(base + co, CH)])
        return carry

    lax.fori_loop(0, NCHUNK, chunk, 0)
    pltpu.sync_copy(acc_v, part_hbm.at[wid])


@functools.partial(
    pl.kernel,
    out_type=(
        jax.ShapeDtypeStruct((N, V), jnp.float32),
        jax.ShapeDtypeStruct((NW, L), jnp.float32),
    ),
    mesh=plsc.VectorSubcoreMesh(core_axis_name="c", subcore_axis_name="s"),
    compiler_params=pltpu.CompilerParams(use_tc_tiling_on_sc=False,
                                         needs_layout_passes=False),
    scratch_types=[
        pltpu.VMEM((RPW,), jnp.int32),
        pltpu.VMEM((RPW,), jnp.int32),
        pltpu.VMEM((V,), jnp.float32),
        pltpu.VMEM((CH, V), jnp.float32),
        pltpu.VMEM((L,), jnp.float32),
        pltpu.SemaphoreType.DMA,
    ],
)
def _sc_gather(table_hbm, in_hbm, tg_hbm, lse_hbm, out_hbm, part_hbm,
               idx_v, tgt_v, lse_v, rows_v, acc_v, gsem):
    _sc_body(table_hbm, in_hbm, tg_hbm, lse_hbm, out_hbm, part_hbm,
             idx_v, tgt_v, lse_v, rows_v, acc_v, gsem)


def kernel(input_sequence, target_sequence, table):
    flat_in = input_sequence.reshape(-1)
    flat_tg = target_sequence.reshape(-1)
    lse = _row_lse(table)
    preds, partials = _sc_gather(table, flat_in, flat_tg,
                                 lse.reshape(-1))
    loss = _finalize(partials)[0, 0]
    return preds, loss


# speedup vs baseline: 1.7044x; 1.0594x over previous
import functools

import jax
import jax.numpy as jnp
from jax import lax
from jax.experimental import pallas as pl
from jax.experimental.pallas import tpu as pltpu
from jax.experimental.pallas import tpu_sc as plsc

V = 1000
N = 51200
NC, NS, L = 2, 16, 16
NW = NC * NS
RPW = N // NW
CH = 32
NCHUNK = RPW // CH
NBUF = 2


def _lse_body(tab_ref, out_ref):
    x = tab_ref[...]
    m = jnp.max(x, axis=1, keepdims=True)
    s = jnp.sum(jnp.exp(x - m), axis=1, keepdims=True)
    out_ref[...] = jnp.log(s) + m


def _row_lse(table):
    return pl.pallas_call(
        _lse_body,
        out_shape=jax.ShapeDtypeStruct((V, 1), jnp.float32),
    )(table)


def _fin_body(p_ref, out_ref):
    out_ref[...] = jnp.full((1, 1), jnp.sum(p_ref[...]) * (1.0 / N),
                            dtype=jnp.float32)


def _finalize(partials):
    return pl.pallas_call(
        _fin_body,
        out_shape=jax.ShapeDtypeStruct((1, 1), jnp.float32),
    )(partials)


def _sc_body(table_hbm, in_hbm, tg_hbm, lse_hbm,
             out_hbm, part_hbm,
             idx_v, tgt_v, lse_v, rows_v, acc_v,
             gsems, ssems):
    wid = lax.axis_index("s") * NC + lax.axis_index("c")
    base = wid * RPW

    pltpu.sync_copy(in_hbm.at[pl.ds(base, RPW)], idx_v)
    pltpu.sync_copy(tg_hbm.at[pl.ds(base, RPW)], tgt_v)
    pltpu.sync_copy(lse_hbm, lse_v)
    acc_v[...] = jnp.zeros((L,), jnp.float32)

    def g_src(c):
        return table_hbm.at[idx_v.at[pl.ds(c * CH, CH)]]

    for b in range(NBUF):
        pltpu.async_copy(g_src(b), rows_v.at[b], gsems[b])

    def iter_body(i, carry):
        for b in range(NBUF):
            c = i * NBUF + b
            co = c * CH
            dst = out_hbm.at[pl.ds(base + co, CH)]
            pltpu.make_async_copy(g_src(c), rows_v.at[b], gsems[b]).wait()
            for g in range(CH // L):
                rid = lax.iota(jnp.int32, L) + (g * L)
                ids = idx_v[pl.ds(co + g * L, L)]
                cid = tgt_v[pl.ds(co + g * L, L)]
                t_el = plsc.load_gather(rows_v.at[b], [rid, cid])
                lse_g = plsc.load_gather(lse_v, [ids])
                acc_v[...] = acc_v[...] + (lse_g - t_el)
            pltpu.async_copy(rows_v.at[b], dst, ssems[b])
            pltpu.make_async_copy(rows_v.at[b], dst, ssems[b]).wait()

            @pl.when(c + NBUF < NCHUNK)
            def _():
                pltpu.async_copy(g_src(c + NBUF), rows_v.at[b], gsems[b])
        return carry

    lax.fori_loop(0, NCHUNK // NBUF, iter_body, 0)
    pltpu.sync_copy(acc_v, part_hbm.at[wid])


@functools.partial(
    pl.kernel,
    out_type=(
        jax.ShapeDtypeStruct((N, V), jnp.float32),
        jax.ShapeDtypeStruct((NW, L), jnp.float32),
    ),
    mesh=plsc.VectorSubcoreMesh(core_axis_name="c", subcore_axis_name="s"),
    compiler_params=pltpu.CompilerParams(use_tc_tiling_on_sc=False,
                                         needs_layout_passes=False),
    scratch_types=[
        pltpu.VMEM((RPW,), jnp.int32),
        pltpu.VMEM((RPW,), jnp.int32),
        pltpu.VMEM((V,), jnp.float32),
        pltpu.VMEM((NBUF, CH, V), jnp.float32),
        pltpu.VMEM((L,), jnp.float32),
        [pltpu.SemaphoreType.DMA] * NBUF,
        [pltpu.SemaphoreType.DMA] * NBUF,
    ],
)
def _sc_gather(table_hbm, in_hbm, tg_hbm, lse_hbm, out_hbm, part_hbm,
               idx_v, tgt_v, lse_v, rows_v, acc_v, gsems, ssems):
    _sc_body(table_hbm, in_hbm, tg_hbm, lse_hbm, out_hbm, part_hbm,
             idx_v, tgt_v, lse_v, rows_v, acc_v, gsems, ssems)


def kernel(input_sequence, target_sequence, table):
    flat_in = input_sequence.reshape(-1)
    flat_tg = target_sequence.reshape(-1)
    lse = _row_lse(table)
    preds, partials = _sc_gather(table, flat_in, flat_tg,
                                 lse.reshape(-1))
    loss = _finalize(partials)[0, 0]
    return preds, loss


# speedup vs baseline: 2.2928x; 1.3452x over previous
import functools

import jax
import jax.numpy as jnp
from jax import lax
from jax.experimental import pallas as pl
from jax.experimental.pallas import tpu as pltpu
from jax.experimental.pallas import tpu_sc as plsc

V = 1000
VH = 896
VT = V - VH
N = 51200
NC, NS, L = 2, 16, 16
NW = NC * NS
RPW = N // NW
CH = 32
NCHUNK = RPW // CH


def _lse_body(tab_ref, out_ref):
    x = tab_ref[...]
    m = jnp.max(x, axis=1, keepdims=True)
    s = jnp.sum(jnp.exp(x - m), axis=1, keepdims=True)
    out_ref[...] = jnp.log(s) + m


def _row_lse(table):
    return pl.pallas_call(
        _lse_body,
        out_shape=jax.ShapeDtypeStruct((V, 1), jnp.float32),
    )(table)


def _fin_body(p_ref, out_ref):
    out_ref[...] = jnp.full((1, 1), jnp.sum(p_ref[...]) * (1.0 / N),
                            dtype=jnp.float32)


def _finalize(partials):
    return pl.pallas_call(
        _fin_body,
        out_shape=jax.ShapeDtypeStruct((1, 1), jnp.float32),
    )(partials)


def _sc_body(tabh_hbm, tabt_hbm, tabf_hbm, lse_hbm, in_hbm, tg_hbm,
             out_hbm, part_hbm,
             idx_v, tgt_v, fidx_v, lse_v, rc_v, rt_v, telem_v, acc_v,
             s1, s2, s3):
    wid = lax.axis_index("s") * NC + lax.axis_index("c")
    base = wid * RPW
    pltpu.sync_copy(in_hbm.at[pl.ds(base, RPW)], idx_v)
    pltpu.sync_copy(tg_hbm.at[pl.ds(base, RPW)], tgt_v)
    pltpu.sync_copy(lse_hbm, lse_v)
    acc_v[...] = jnp.zeros((L,), jnp.float32)

    def fx(i, carry):
        o = i * L
        fidx_v[pl.ds(o, L)] = idx_v[pl.ds(o, L)] * V + tgt_v[pl.ds(o, L)]
        return carry

    lax.fori_loop(0, RPW // L, fx, 0)

    def chunk(c, carry):
        co = c * CH
        d1 = pltpu.async_copy(
            tabh_hbm.at[idx_v.at[pl.ds(co, CH)]],
            rc_v.at[:, pl.ds(0, VH)], s1)
        d2 = pltpu.async_copy(
            tabt_hbm.at[idx_v.at[pl.ds(co, CH)]], rt_v, s2)
        d3 = pltpu.async_copy(
            tabf_hbm.at[fidx_v.at[pl.ds(co, CH)]], telem_v, s3)
        d1.wait()
        d2.wait()
        for r in range(CH):
            for o in (0, 16, 32, 48, 64, 80):
                rc_v[r, pl.ds(VH + o, 16)] = rt_v[r, pl.ds(o, 16)]
            rc_v[r, pl.ds(V - 16, 16)] = rt_v[r, pl.ds(VT - 16, 16)]
        pltpu.sync_copy(rc_v, out_hbm.at[pl.ds(base + co, CH)])
        d3.wait()
        for g in range(CH // L):
            ids = idx_v[pl.ds(co + g * L, L)]
            lse_g = plsc.load_gather(lse_v, [ids])
            acc_v[...] = acc_v[...] + (lse_g - telem_v[pl.ds(g * L, L)])
        return carry

    lax.fori_loop(0, NCHUNK, chunk, 0)
    pltpu.sync_copy(acc_v, part_hbm.at[wid])


@functools.partial(
    pl.kernel,
    out_type=(
        jax.ShapeDtypeStruct((N, V), jnp.float32),
        jax.ShapeDtypeStruct((NW, L), jnp.float32),
    ),
    mesh=plsc.VectorSubcoreMesh(core_axis_name="c", subcore_axis_name="s"),
    compiler_params=pltpu.CompilerParams(use_tc_tiling_on_sc=True,
                                         needs_layout_passes=False),
    scratch_types=[
        pltpu.VMEM((RPW,), jnp.int32),
        pltpu.VMEM((RPW,), jnp.int32),
        pltpu.VMEM((RPW,), jnp.int32),
        pltpu.VMEM((V,), jnp.float32),
        pltpu.VMEM((CH, V), jnp.float32),
        pltpu.VMEM((CH, 128), jnp.float32),
        pltpu.VMEM((CH,), jnp.float32),
        pltpu.VMEM((L,), jnp.float32),
        pltpu.SemaphoreType.DMA,
        pltpu.SemaphoreType.DMA,
        pltpu.SemaphoreType.DMA,
    ],
)
def _sc_gather(tabh_hbm, tabt_hbm, tabf_hbm, lse_hbm, in_hbm, tg_hbm,
               out_hbm, part_hbm,
               idx_v, tgt_v, fidx_v, lse_v, rc_v, rt_v, telem_v, acc_v,
               s1, s2, s3):
    _sc_body(tabh_hbm, tabt_hbm, tabf_hbm, lse_hbm, in_hbm, tg_hbm,
             out_hbm, part_hbm,
             idx_v, tgt_v, fidx_v, lse_v, rc_v, rt_v, telem_v, acc_v,
             s1, s2, s3)


def kernel(input_sequence, target_sequence, table):
    flat_in = input_sequence.reshape(-1)
    flat_tg = target_sequence.reshape(-1)
    tabh = table[:, :VH]
    tabt = jnp.pad(table[:, VH:], ((0, 0), (0, 128 - VT)))
    tabf = jnp.pad(table.reshape(-1), (0, 8))
    lse = _row_lse(table)
    preds, partials = _sc_gather(tabh, tabt, tabf, lse.reshape(-1),
                                 flat_in, flat_tg)
    loss = _finalize(partials)[0, 0]
    return preds, loss
